# baseline (speedup 1.0000x reference)
import jax
import jax.numpy as jnp
from jax.experimental import pallas as pl
from jax.experimental.pallas import tpu as pltpu

_ROWS = 65536
_D = 32
_CBLK = 4096


def _matmul_t_block(xt_ref, w_ref, b_ref, o_ref):
    yt = jax.lax.dot_general(
        w_ref[...], xt_ref[...],
        dimension_numbers=(((0,), (0,)), ((), ())),
        preferred_element_type=jnp.float32,
    )
    o_ref[...] = yt + jax.lax.broadcast_in_dim(b_ref[...], (_D, _CBLK), (0,))


def kernel(x, W, b):
    xt = x.T
    yt = pl.pallas_call(
        _matmul_t_block,
        grid=(_ROWS // _CBLK,),
        in_specs=[
            pl.BlockSpec((_D, _CBLK), lambda i: (0, i)),
            pl.BlockSpec((_D, _D), lambda i: (0, 0)),
            pl.BlockSpec((_D,), lambda i: (0,)),
        ],
        out_specs=pl.BlockSpec((_D, _CBLK), lambda i: (0, i)),
        out_shape=jax.ShapeDtypeStruct((_D, _ROWS), jnp.float32),
        compiler_params=pltpu.CompilerParams(
            dimension_semantics=("parallel",),
        ),
    )(xt, W, b)
    return yt.T


# speedup vs baseline: 1.8022x; 1.8022x over previous
import jax
import jax.numpy as jnp
from jax.experimental import pallas as pl
from jax.experimental.pallas import tpu as pltpu

_ROWS = 65536
_D = 32
_CBLK = 16384


def _matmul_t_block(xt_ref, w_ref, b_ref, o_ref):
    yt = jax.lax.dot_general(
        w_ref[...], xt_ref[...],
        dimension_numbers=(((0,), (0,)), ((), ())),
        preferred_element_type=jnp.float32,
    )
    o_ref[...] = yt + jax.lax.broadcast_in_dim(b_ref[...], (_D, _CBLK), (0,))


def kernel(x, W, b):
    xt = x.T
    yt = pl.pallas_call(
        _matmul_t_block,
        grid=(_ROWS // _CBLK,),
        in_specs=[
            pl.BlockSpec((_D, _CBLK), lambda i: (0, i)),
            pl.BlockSpec((_D, _D), lambda i: (0, 0)),
            pl.BlockSpec((_D,), lambda i: (0,)),
        ],
        out_specs=pl.BlockSpec((_D, _CBLK), lambda i: (0, i)),
        out_shape=jax.ShapeDtypeStruct((_D, _ROWS), jnp.float32),
        compiler_params=pltpu.CompilerParams(
            dimension_semantics=("parallel",),
        ),
    )(xt, W, b)
    return yt.T


# speedup vs baseline: 2.2229x; 1.2335x over previous
import jax
import jax.numpy as jnp
from jax.experimental import pallas as pl
from jax.experimental.pallas import tpu as pltpu

_ROWS = 65536
_D = 32
_CBLK = 32768


def _matmul_t_block(xt_ref, w_ref, b_ref, o_ref):
    yt = jax.lax.dot_general(
        w_ref[...], xt_ref[...],
        dimension_numbers=(((0,), (0,)), ((), ())),
        preferred_element_type=jnp.float32,
    )
    o_ref[...] = yt + jax.lax.broadcast_in_dim(b_ref[...], (_D, _CBLK), (0,))


def kernel(x, W, b):
    xt = x.T
    yt = pl.pallas_call(
        _matmul_t_block,
        grid=(_ROWS // _CBLK,),
        in_specs=[
            pl.BlockSpec((_D, _CBLK), lambda i: (0, i)),
            pl.BlockSpec((_D, _D), lambda i: (0, 0)),
            pl.BlockSpec((_D,), lambda i: (0,)),
        ],
        out_specs=pl.BlockSpec((_D, _CBLK), lambda i: (0, i)),
        out_shape=jax.ShapeDtypeStruct((_D, _ROWS), jnp.float32),
        compiler_params=pltpu.CompilerParams(
            dimension_semantics=("parallel",),
        ),
    )(xt, W, b)
    return yt.T


# speedup vs baseline: 2.2392x; 1.0073x over previous
import jax
import jax.numpy as jnp
from jax.experimental import pallas as pl
from jax.experimental.pallas import tpu as pltpu

_ROWS = 65536
_D = 32
_CBLK = 32768


def _matmul_t_block(xt_ref, w_ref, b_ref, o_ref):
    yt = jax.lax.dot_general(
        w_ref[...], xt_ref[...],
        dimension_numbers=(((0,), (0,)), ((), ())),
        preferred_element_type=jnp.float32,
    )
    o_ref[...] = yt + jax.lax.broadcast_in_dim(b_ref[...], (_D, _CBLK), (0,))


def kernel(x, W, b):
    xt = x.T
    yt = pl.pallas_call(
        _matmul_t_block,
        grid=(_ROWS // _CBLK,),
        in_specs=[
            pl.BlockSpec((_D, _CBLK), lambda i: (0, i)),
            pl.BlockSpec((_D, _D), lambda i: (0, 0)),
            pl.BlockSpec((_D,), lambda i: (0,)),
        ],
        out_specs=pl.BlockSpec((_D, _CBLK), lambda i: (0, i)),
        out_shape=jax.ShapeDtypeStruct((_D, _ROWS), jnp.float32),
        compiler_params=pltpu.CompilerParams(
            dimension_semantics=("arbitrary",),
        ),
    )(xt, W, b)
    return yt.T
